# baseline (speedup 1.0000x reference)
import functools

import jax
import jax.numpy as jnp
from jax import lax
from jax.experimental import pallas as pl
from jax.experimental.pallas import tpu as pltpu
from jax.experimental.pallas import tpu_sc as plsc

T = 4096
L = 16
NCHUNK = T // L
BIG = T + 10
PAD = 128


def _mask_body(labels_hbm, out_hbm, press_v, rel_v, end_v, out_v,
               press_sem, rel_sem, out_sem):
    wid = lax.axis_index("s") * 2 + lax.axis_index("c")
    b = wid // 2
    f = wid % 2
    H = T // 2

    zeros = jnp.zeros((L,), jnp.float32)
    press_v[pl.ds(PAD - L, L)] = zeros
    press_v[pl.ds(PAD + T, L)] = zeros
    rel_v[pl.ds(PAD + T, L)] = zeros
    press_cp = pltpu.async_copy(
        labels_hbm.at[b, 2 * f], press_v.at[pl.ds(PAD, T)], press_sem
    )
    rel_lo_cp = pltpu.async_copy(
        labels_hbm.at[b, 2 * f + 1, pl.ds(0, H)],
        rel_v.at[pl.ds(PAD, H)], rel_sem
    )
    pltpu.sync_copy(labels_hbm.at[b, 2 * f + 1, pl.ds(H, H)],
                    rel_v.at[pl.ds(PAD + H, H)])

    lanes = lax.broadcasted_iota(jnp.int32, (L,), 0)

    def bwd(i, carry):
        c = NCHUNK - 1 - i
        base = PAD + c * L
        rel_c = rel_v[pl.ds(base, L)]
        rel_n = rel_v[pl.ds(base + 1, L)]
        t_idx = c * L + lanes
        z = jnp.where(rel_n - rel_c > 0, t_idx + 1, BIG)
        p = plsc.cummax(-lax.rev(z, (0,)))
        next_after = -lax.rev(jnp.maximum(p, -carry), (0,))
        end_v[pl.ds(c * L, L)] = jnp.minimum(next_after + 8, T)
        return jnp.minimum(carry, jnp.min(z))

    carry_hi = plsc.parallel_loop(
        0, NCHUNK // 2, carry=jnp.int32(BIG), unroll=4)(bwd)
    rel_lo_cp.wait()
    plsc.parallel_loop(
        NCHUNK // 2, NCHUNK, carry=carry_hi, unroll=4)(bwd)
    press_cp.wait()

    def fwd(c, carry):
        base = PAD + c * L
        press_c = press_v[pl.ds(base, L)]
        press_p = press_v[pl.ds(base - 1, L)]
        cand = jnp.where(press_c - press_p > 0, end_v[pl.ds(c * L, L)], 0)
        e = jnp.maximum(plsc.cummax(cand), carry)
        t_idx = c * L + lanes
        out_v[pl.ds(c * L, L)] = jnp.where(e > t_idx, 1.0, 0.0)
        return jnp.maximum(carry, jnp.max(cand))

    carry_lo = plsc.parallel_loop(
        0, NCHUNK // 2, carry=jnp.int32(0), unroll=4)(fwd)
    out_lo_cp = pltpu.async_copy(
        out_v.at[pl.ds(0, H)], out_hbm.at[b, f, pl.ds(0, H)], out_sem
    )
    plsc.parallel_loop(
        NCHUNK // 2, NCHUNK, carry=carry_lo, unroll=4)(fwd)
    pltpu.sync_copy(out_v.at[pl.ds(H, H)], out_hbm.at[b, f, pl.ds(H, H)])
    out_lo_cp.wait()


@jax.jit
def kernel(gesture_labels):
    B = gesture_labels.shape[0]
    mesh = plsc.VectorSubcoreMesh(
        core_axis_name="c", subcore_axis_name="s", num_cores=2, num_subcores=16
    )
    call = pl.kernel(
        _mask_body,
        out_type=jax.ShapeDtypeStruct((B, 2, T), jnp.float32),
        mesh=mesh,
        scratch_types=[
            pltpu.VMEM((PAD + T + PAD,), jnp.float32),
            pltpu.VMEM((PAD + T + PAD,), jnp.float32),
            pltpu.VMEM((T,), jnp.int32),
            pltpu.VMEM((T,), jnp.float32),
            pltpu.SemaphoreType.DMA,
            pltpu.SemaphoreType.DMA,
            pltpu.SemaphoreType.DMA,
        ],
        compiler_params=pltpu.CompilerParams(
            needs_layout_passes=False,
        ),
    )
    return call(gesture_labels)


# speedup vs baseline: 1.0296x; 1.0296x over previous
import functools

import jax
import jax.numpy as jnp
from jax import lax
from jax.experimental import pallas as pl
from jax.experimental.pallas import tpu as pltpu
from jax.experimental.pallas import tpu_sc as plsc

T = 4096
L = 16
NCHUNK = T // L
BIG = T + 10
PAD = 128


def _mask_body(labels_hbm, out_hbm, press_v, rel_v, end_v, out_v, dma_sem):
    wid = lax.axis_index("s") * 2 + lax.axis_index("c")
    b = wid // 2
    f = wid % 2

    zeros = jnp.zeros((L,), jnp.float32)
    press_v[pl.ds(PAD - L, L)] = zeros
    press_v[pl.ds(PAD + T, L)] = zeros
    rel_v[pl.ds(PAD + T, L)] = zeros
    press_cp = pltpu.async_copy(
        labels_hbm.at[b, 2 * f], press_v.at[pl.ds(PAD, T)], dma_sem
    )
    pltpu.sync_copy(labels_hbm.at[b, 2 * f + 1], rel_v.at[pl.ds(PAD, T)])

    lanes = lax.broadcasted_iota(jnp.int32, (L,), 0)

    @plsc.parallel_loop(0, NCHUNK, carry=jnp.int32(BIG), unroll=4)
    def bwd(i, carry):
        c = NCHUNK - 1 - i
        base = PAD + c * L
        rel_c = rel_v[pl.ds(base, L)]
        rel_n = rel_v[pl.ds(base + 1, L)]
        t_idx = c * L + lanes
        z = jnp.where(rel_n - rel_c > 0, t_idx + 1, BIG)
        p = plsc.cummax(-lax.rev(z, (0,)))
        next_after = -lax.rev(jnp.maximum(p, -carry), (0,))
        end_v[pl.ds(c * L, L)] = jnp.minimum(next_after + 8, T)
        return jnp.minimum(carry, jnp.min(z))

    press_cp.wait()

    @plsc.parallel_loop(0, NCHUNK, carry=jnp.int32(0), unroll=4)
    def fwd(c, carry):
        base = PAD + c * L
        press_c = press_v[pl.ds(base, L)]
        press_p = press_v[pl.ds(base - 1, L)]
        cand = jnp.where(press_c - press_p > 0, end_v[pl.ds(c * L, L)], 0)
        e = jnp.maximum(plsc.cummax(cand), carry)
        t_idx = c * L + lanes
        out_v[pl.ds(c * L, L)] = jnp.where(e > t_idx, 1.0, 0.0)
        return jnp.maximum(carry, jnp.max(cand))

    pltpu.sync_copy(out_v, out_hbm.at[b, f])


@jax.jit
def kernel(gesture_labels):
    B = gesture_labels.shape[0]
    mesh = plsc.VectorSubcoreMesh(
        core_axis_name="c", subcore_axis_name="s", num_cores=2, num_subcores=16
    )
    call = pl.kernel(
        _mask_body,
        out_type=jax.ShapeDtypeStruct((B, 2, T), jnp.float32),
        mesh=mesh,
        scratch_types=[
            pltpu.VMEM((PAD + T + PAD,), jnp.float32),
            pltpu.VMEM((PAD + T + PAD,), jnp.float32),
            pltpu.VMEM((T,), jnp.int32),
            pltpu.VMEM((T,), jnp.float32),
            pltpu.SemaphoreType.DMA,
        ],
        compiler_params=pltpu.CompilerParams(
            needs_layout_passes=False,
        ),
    )
    return call(gesture_labels)
